# Initial kernel scaffold; baseline (speedup 1.0000x reference)
import jax
import jax.numpy as jnp
from jax.experimental import pallas as pl


def kernel(input, codebooks, weights, random_vector):
    raise NotImplementedError("write your pallas kernel here")


# speedup vs baseline: 2.2112x; 2.2112x over previous
import functools

import jax
import jax.numpy as jnp
from jax.experimental import pallas as pl
from jax.experimental.pallas import tpu as pltpu


def _nsvq_body(x_ref, cb_ref, w_ref, rv_ref, o_ref, dmin_ref):
    k = pl.program_id(1)
    nk = pl.num_programs(1)

    wa = jnp.abs(w_ref[0, :])
    wx = x_ref[...] * wa[None, :]
    wc = cb_ref[...] * wa[None, :]
    in2 = jnp.sum(wx * wx, axis=1, keepdims=True)
    cb2 = jnp.sum(wc * wc, axis=1)
    scores = jnp.dot(wx, wc.T, preferred_element_type=jnp.float32)
    dist = (in2 - 2.0 * scores) + cb2[None, :]
    m = jnp.min(dist, axis=1, keepdims=True)

    @pl.when(k == 0)
    def _init():
        dmin_ref[...] = m

    @pl.when(k > 0)
    def _acc():
        dmin_ref[...] = jnp.minimum(dmin_ref[...], m)

    @pl.when(k == nk - 1)
    def _epilogue():
        eps = 1e-12
        rv = rv_ref[...]
        nrand = jnp.sqrt(jnp.sum(rv * rv, axis=1, keepdims=True))
        nbest = jnp.sqrt(jnp.maximum(dmin_ref[...], 0.0))
        scale = nbest / nrand + eps
        o_ref[...] = x_ref[...] + scale * rv * (1.0 / (wa[None, :] + eps))


@jax.jit
def kernel(input, codebooks, weights, random_vector):
    n, d = input.shape
    kk = codebooks.shape[0]
    bn = min(2048, n)
    bk = min(1024, kk)
    w2d = weights.reshape(1, d)
    grid = (n // bn, kk // bk)
    return pl.pallas_call(
        _nsvq_body,
        grid=grid,
        in_specs=[
            pl.BlockSpec((bn, d), lambda i, j: (i, 0)),
            pl.BlockSpec((bk, d), lambda i, j: (j, 0)),
            pl.BlockSpec((1, d), lambda i, j: (0, 0)),
            pl.BlockSpec((bn, d), lambda i, j: (i, 0)),
        ],
        out_specs=pl.BlockSpec((bn, d), lambda i, j: (i, 0)),
        out_shape=jax.ShapeDtypeStruct((n, d), jnp.float32),
        scratch_shapes=[pltpu.VMEM((bn, 1), jnp.float32)],
        compiler_params=pltpu.CompilerParams(
            dimension_semantics=("arbitrary", "arbitrary"),
        ),
    )(input, codebooks, w2d, random_vector)


# speedup vs baseline: 2.2563x; 1.0204x over previous
import functools

import jax
import jax.numpy as jnp
from jax.experimental import pallas as pl
from jax.experimental.pallas import tpu as pltpu


def _nsvq_body(x_ref, cb_ref, w_ref, rv_ref, o_ref, dmin_ref):
    k = pl.program_id(1)
    nk = pl.num_programs(1)

    wa = jnp.abs(w_ref[0, :])
    wx = x_ref[...] * wa[None, :]
    wc = cb_ref[...] * wa[None, :]
    in2 = jnp.sum(wx * wx, axis=1, keepdims=True)
    cb2 = jnp.sum(wc * wc, axis=1)
    scores = jnp.dot(wx.astype(jnp.bfloat16), wc.astype(jnp.bfloat16).T,
                     preferred_element_type=jnp.float32)
    dist = (in2 - 2.0 * scores) + cb2[None, :]
    m = jnp.min(dist, axis=1, keepdims=True)

    @pl.when(k == 0)
    def _init():
        dmin_ref[...] = m

    @pl.when(k > 0)
    def _acc():
        dmin_ref[...] = jnp.minimum(dmin_ref[...], m)

    @pl.when(k == nk - 1)
    def _epilogue():
        eps = 1e-12
        rv = rv_ref[...]
        nrand = jnp.sqrt(jnp.sum(rv * rv, axis=1, keepdims=True))
        nbest = jnp.sqrt(jnp.maximum(dmin_ref[...], 0.0))
        scale = nbest / nrand + eps
        o_ref[...] = x_ref[...] + scale * rv * (1.0 / (wa[None, :] + eps))


@jax.jit
def kernel(input, codebooks, weights, random_vector):
    n, d = input.shape
    kk = codebooks.shape[0]
    bn = min(2048, n)
    bk = min(1024, kk)
    w2d = weights.reshape(1, d)
    grid = (n // bn, kk // bk)
    return pl.pallas_call(
        _nsvq_body,
        grid=grid,
        in_specs=[
            pl.BlockSpec((bn, d), lambda i, j: (i, 0)),
            pl.BlockSpec((bk, d), lambda i, j: (j, 0)),
            pl.BlockSpec((1, d), lambda i, j: (0, 0)),
            pl.BlockSpec((bn, d), lambda i, j: (i, 0)),
        ],
        out_specs=pl.BlockSpec((bn, d), lambda i, j: (i, 0)),
        out_shape=jax.ShapeDtypeStruct((n, d), jnp.float32),
        scratch_shapes=[pltpu.VMEM((bn, 1), jnp.float32)],
        compiler_params=pltpu.CompilerParams(
            dimension_semantics=("arbitrary", "arbitrary"),
        ),
    )(input, codebooks, w2d, random_vector)
